# baseline (speedup 1.0000x reference)
import functools

import jax
import jax.numpy as jnp
from jax import lax
from jax.experimental import pallas as pl
from jax.experimental.pallas import tpu as pltpu
from jax.experimental.pallas import tpu_sc as plsc

_N_USERS = 1000000
_N_ITEMS = 100000
_N_GENRES = 26
_K = 16
_B = 16384

_NUM_WORKERS = 32
_NIDX = _B * _K
_BPW = _NIDX // _NUM_WORKERS


def _sc_gather(p_flat, q_flat, uidx, iidx):
    mesh = plsc.VectorSubcoreMesh(core_axis_name="c", subcore_axis_name="s")

    @functools.partial(
        pl.kernel,
        mesh=mesh,
        out_type=(
            jax.ShapeDtypeStruct((_NIDX,), jnp.float32),
            jax.ShapeDtypeStruct((_NIDX,), jnp.float32),
        ),
        scratch_types=[
            pltpu.VMEM((_BPW,), jnp.int32),
            pltpu.VMEM((_BPW,), jnp.float32),
            pltpu.VMEM((_BPW,), jnp.int32),
            pltpu.VMEM((_BPW,), jnp.float32),
            pltpu.SemaphoreType.DMA,
            pltpu.SemaphoreType.DMA,
        ],
    )
    def gather_kernel(p_hbm, q_hbm, ui_hbm, ii_hbm, pu_hbm, qi_hbm,
                      ui_v, pu_v, ii_v, qi_v, sem_p, sem_q):
        wid = lax.axis_index("s") * 2 + lax.axis_index("c")
        base = wid * _BPW
        pltpu.sync_copy(ui_hbm.at[pl.ds(base, _BPW)], ui_v)
        pltpu.sync_copy(ii_hbm.at[pl.ds(base, _BPW)], ii_v)
        cp_p = pltpu.async_copy(p_hbm.at[ui_v], pu_v, sem_p)
        cp_q = pltpu.async_copy(q_hbm.at[ii_v], qi_v, sem_q)
        cp_p.wait()
        cp_q.wait()
        pltpu.sync_copy(pu_v, pu_hbm.at[pl.ds(base, _BPW)])
        pltpu.sync_copy(qi_v, qi_hbm.at[pl.ds(base, _BPW)])

    return gather_kernel(p_flat, q_flat, uidx, iidx)


def _combine_body(pu_ref, qi_ref, g_ref, wgt_ref, o_ref):
    qg = jnp.dot(g_ref[...], wgt_ref[...], preferred_element_type=jnp.float32)
    o_ref[...] = jnp.sum(pu_ref[...] * (qi_ref[...] + qg), axis=1,
                         keepdims=True)


def _tc_combine(pu, qi, genres, wg_t):
    blk = 2048
    grid = (_B // blk,)
    return pl.pallas_call(
        _combine_body,
        out_shape=jax.ShapeDtypeStruct((_B, 1), jnp.float32),
        grid=grid,
        in_specs=[
            pl.BlockSpec((blk, _K), lambda i: (i, 0)),
            pl.BlockSpec((blk, _K), lambda i: (i, 0)),
            pl.BlockSpec((blk, _N_GENRES), lambda i: (i, 0)),
            pl.BlockSpec((_N_GENRES, _K), lambda i: (0, 0)),
        ],
        out_specs=pl.BlockSpec((blk, 1), lambda i: (i, 0)),
    )(pu, qi, genres, wg_t)


def kernel(user_ids, item_ids, genres_one_hot, P_w, Q_w):
    ku = jnp.arange(_K, dtype=jnp.int32) * _N_USERS
    kq = jnp.arange(_K, dtype=jnp.int32) * (_N_ITEMS + _N_GENRES)
    uidx = (user_ids.astype(jnp.int32)[:, None] + ku[None, :]).reshape(-1)
    iidx = (item_ids.astype(jnp.int32)[:, None] + kq[None, :]).reshape(-1)
    p_flat = P_w.reshape(-1)
    q_flat = Q_w.reshape(-1)
    wg_t = Q_w[:, _N_ITEMS:].T

    pu_flat, qi_flat = _sc_gather(p_flat, q_flat, uidx, iidx)
    pu = pu_flat.reshape(_B, _K)
    qi = qi_flat.reshape(_B, _K)
    return _tc_combine(pu, qi, genres_one_hot, wg_t)


# speedup vs baseline: 2.6723x; 2.6723x over previous
import functools

import jax
import jax.numpy as jnp
from jax import lax
from jax.experimental import pallas as pl
from jax.experimental.pallas import tpu as pltpu
from jax.experimental.pallas import tpu_sc as plsc

_N_USERS = 1000000
_N_ITEMS = 100000
_N_GENRES = 26
_K = 16
_B = 16384

_NUM_WORKERS = 32
_BPW = _B // _NUM_WORKERS


def _sc_gather(p_t, q_t, user_ids, item_ids):
    mesh = plsc.VectorSubcoreMesh(core_axis_name="c", subcore_axis_name="s")

    @functools.partial(
        pl.kernel,
        mesh=mesh,
        out_type=(
            jax.ShapeDtypeStruct((_B, _K), jnp.float32),
            jax.ShapeDtypeStruct((_B, _K), jnp.float32),
        ),
        compiler_params=pltpu.CompilerParams(use_tc_tiling_on_sc=False),
        scratch_types=[
            pltpu.VMEM((_BPW,), jnp.int32),
            pltpu.VMEM((_BPW, _K), jnp.float32),
            pltpu.VMEM((_BPW,), jnp.int32),
            pltpu.VMEM((_BPW, _K), jnp.float32),
            pltpu.SemaphoreType.DMA,
            pltpu.SemaphoreType.DMA,
        ],
    )
    def gather_kernel(p_hbm, q_hbm, ui_hbm, ii_hbm, pu_hbm, qi_hbm,
                      ui_v, pu_v, ii_v, qi_v, sem_p, sem_q):
        wid = lax.axis_index("s") * 2 + lax.axis_index("c")
        base = wid * _BPW
        pltpu.sync_copy(ui_hbm.at[pl.ds(base, _BPW)], ui_v)
        pltpu.sync_copy(ii_hbm.at[pl.ds(base, _BPW)], ii_v)
        cp_p = pltpu.async_copy(p_hbm.at[ui_v], pu_v, sem_p)
        cp_q = pltpu.async_copy(q_hbm.at[ii_v], qi_v, sem_q)
        cp_p.wait()
        cp_q.wait()
        pltpu.sync_copy(pu_v, pu_hbm.at[pl.ds(base, _BPW)])
        pltpu.sync_copy(qi_v, qi_hbm.at[pl.ds(base, _BPW)])

    return gather_kernel(p_t, q_t, user_ids, item_ids)


def _combine_body(pu_ref, qi_ref, g_ref, wgt_ref, o_ref):
    qg = jnp.dot(g_ref[...], wgt_ref[...], preferred_element_type=jnp.float32)
    o_ref[...] = jnp.sum(pu_ref[...] * (qi_ref[...] + qg), axis=1,
                         keepdims=True)


def _tc_combine(pu, qi, genres, wg_t):
    blk = 2048
    grid = (_B // blk,)
    return pl.pallas_call(
        _combine_body,
        out_shape=jax.ShapeDtypeStruct((_B, 1), jnp.float32),
        grid=grid,
        in_specs=[
            pl.BlockSpec((blk, _K), lambda i: (i, 0)),
            pl.BlockSpec((blk, _K), lambda i: (i, 0)),
            pl.BlockSpec((blk, _N_GENRES), lambda i: (i, 0)),
            pl.BlockSpec((_N_GENRES, _K), lambda i: (0, 0)),
        ],
        out_specs=pl.BlockSpec((blk, 1), lambda i: (i, 0)),
    )(pu, qi, genres, wg_t)


def kernel(user_ids, item_ids, genres_one_hot, P_w, Q_w):
    p_t = P_w.T
    q_t = Q_w.T
    wg_t = q_t[_N_ITEMS:, :]

    pu, qi = _sc_gather(p_t, q_t, user_ids.astype(jnp.int32),
                        item_ids.astype(jnp.int32))
    return _tc_combine(pu, qi, genres_one_hot, wg_t)


# speedup vs baseline: 4.5678x; 1.7093x over previous
import functools

import jax
import jax.numpy as jnp
from jax import lax
from jax.experimental import pallas as pl
from jax.experimental.pallas import tpu as pltpu
from jax.experimental.pallas import tpu_sc as plsc

_N_USERS = 1000000
_N_ITEMS = 100000
_N_GENRES = 26
_K = 16
_B = 16384

_NUM_WORKERS = 32
_NPIECES = 8
_KPP = _K // _NPIECES
_NIDX = _B * _K
_BPW = _NIDX // _NUM_WORKERS


def _sc_gather(p_pieces, q_flat, uidx, iidx):
    mesh = plsc.VectorSubcoreMesh(core_axis_name="c", subcore_axis_name="s")

    @functools.partial(
        pl.kernel,
        mesh=mesh,
        out_type=(
            jax.ShapeDtypeStruct((_NIDX,), jnp.float32),
            jax.ShapeDtypeStruct((_NIDX,), jnp.float32),
        ),
        scratch_types=[
            pltpu.VMEM((_BPW,), jnp.int32),
            pltpu.VMEM((_BPW,), jnp.float32),
            pltpu.VMEM((_BPW,), jnp.int32),
            pltpu.VMEM((_BPW,), jnp.float32),
            pltpu.SemaphoreType.DMA,
            pltpu.SemaphoreType.DMA,
        ],
    )
    def gather_kernel(*refs):
        p_refs = refs[:_NPIECES]
        (q_hbm, ui_hbm, ii_hbm, pu_hbm, qi_hbm,
         ui_v, pu_v, ii_v, qi_v, sem_p, sem_q) = refs[_NPIECES:]
        wid = lax.axis_index("s") * 2 + lax.axis_index("c")
        base = wid * _BPW
        piece = wid // (_NUM_WORKERS // _NPIECES)
        pltpu.sync_copy(ui_hbm.at[pl.ds(base, _BPW)], ui_v)
        pltpu.sync_copy(ii_hbm.at[pl.ds(base, _BPW)], ii_v)
        for pc in range(_NPIECES):
            @pl.when(piece == pc)
            def _():
                cp = pltpu.async_copy(p_refs[pc].at[ui_v], pu_v, sem_p)
                cp.wait()
        cp_q = pltpu.async_copy(q_hbm.at[ii_v], qi_v, sem_q)
        cp_q.wait()
        pltpu.sync_copy(pu_v, pu_hbm.at[pl.ds(base, _BPW)])
        pltpu.sync_copy(qi_v, qi_hbm.at[pl.ds(base, _BPW)])

    return gather_kernel(*p_pieces, q_flat, uidx, iidx)


def _combine_body(pu_ref, qi_ref, g_ref, wgt_ref, o_ref):
    qg = jnp.dot(g_ref[...], wgt_ref[...], preferred_element_type=jnp.float32)
    o_ref[...] = jnp.sum(pu_ref[...] * (qi_ref[...] + qg), axis=1,
                         keepdims=True)


def _tc_combine(pu, qi, genres, wg_t):
    blk = 2048
    grid = (_B // blk,)
    return pl.pallas_call(
        _combine_body,
        out_shape=jax.ShapeDtypeStruct((_B, 1), jnp.float32),
        grid=grid,
        in_specs=[
            pl.BlockSpec((blk, _K), lambda i: (i, 0)),
            pl.BlockSpec((blk, _K), lambda i: (i, 0)),
            pl.BlockSpec((blk, _N_GENRES), lambda i: (i, 0)),
            pl.BlockSpec((_N_GENRES, _K), lambda i: (0, 0)),
        ],
        out_specs=pl.BlockSpec((blk, 1), lambda i: (i, 0)),
    )(pu, qi, genres, wg_t)


def kernel(user_ids, item_ids, genres_one_hot, P_w, Q_w):
    ku = jnp.arange(_K, dtype=jnp.int32) % _KPP * _N_USERS
    kq = jnp.arange(_K, dtype=jnp.int32) * (_N_ITEMS + _N_GENRES)
    uidx = (ku[:, None] + user_ids.astype(jnp.int32)[None, :]).reshape(-1)
    iidx = (item_ids.astype(jnp.int32)[:, None] + kq[None, :]).reshape(-1)
    p_pieces = [P_w[pc * _KPP:(pc + 1) * _KPP].reshape(-1)
                for pc in range(_NPIECES)]
    q_flat = Q_w.reshape(-1)
    wg_t = Q_w[:, _N_ITEMS:].T

    puk_flat, qi_flat = _sc_gather(p_pieces, q_flat, uidx, iidx)
    pu = puk_flat.reshape(_K, _B).T
    qi = qi_flat.reshape(_B, _K)
    return _tc_combine(pu, qi, genres_one_hot, wg_t)
